# baseline (speedup 1.0000x reference)
import functools

import jax
import jax.numpy as jnp
from jax import lax
from jax.experimental import pallas as pl
from jax.experimental.pallas import tpu as pltpu
from jax.experimental.pallas import tpu_sc as plsc

E = 8
B = 2048
LATENT = 128
HIDDEN = 512
RES = (3, 32, 32)
OUT = RES[0] * RES[1] * RES[2]

BLK = 64
NB = B // BLK + E
R = NB * BLK
NW = 32
GROWS = R // NW
UCH = 16
UK = B // (NW * UCH)


def _wid():
    return lax.axis_index("s") * 2 + lax.axis_index("c")


@functools.lru_cache(maxsize=1)
def _sc_kernels():
    mesh = plsc.VectorSubcoreMesh(core_axis_name="c", subcore_axis_name="s")

    @functools.partial(
        pl.kernel,
        mesh=mesh,
        out_type=jax.ShapeDtypeStruct((B, OUT), jnp.float32),
        scratch_types=[
            pltpu.VMEM((UCH * UK,), jnp.int32),
            pltpu.VMEM((UCH, OUT), jnp.float32),
            pltpu.VMEM((UCH, OUT), jnp.float32),
            pltpu.SemaphoreType.DMA,
            pltpu.SemaphoreType.DMA,
        ],
    )
    def sc_unsort(y_hbm, pos_hbm, out_hbm, idx_v, buf0, buf1, sem0, sem1):
        base = _wid() * (UCH * UK)
        pltpu.sync_copy(pos_hbm.at[pl.ds(base, UCH * UK)], idx_v)
        bufs = (buf0, buf1)
        sems = (sem0, sem1)
        cps = [None, None]
        for k in range(UK):
            cps[k % 2] = pltpu.async_copy(
                y_hbm.at[idx_v.at[pl.ds(k * UCH, UCH)]], bufs[k % 2],
                sems[k % 2])
            if k > 0:
                cps[(k - 1) % 2].wait()
                pltpu.sync_copy(
                    bufs[(k - 1) % 2],
                    out_hbm.at[pl.ds(base + (k - 1) * UCH, UCH)])
        cps[(UK - 1) % 2].wait()
        pltpu.sync_copy(bufs[(UK - 1) % 2],
                        out_hbm.at[pl.ds(base + (UK - 1) * UCH, UCH)])

    return sc_unsort


def _mm_body(be_ref, gidx_ref, x_ref, w1_ref, b1_ref, w2_ref, b2_ref, w3_ref,
             b3_ref, o_ref):
    gi = gidx_ref[0]
    sel = (lax.broadcasted_iota(jnp.int32, (BLK, B), 1) == gi)
    x = jnp.dot(sel.astype(jnp.float32), x_ref[...],
                preferred_element_type=jnp.float32)
    h = jnp.maximum(
        jnp.dot(x, w1_ref[0], preferred_element_type=jnp.float32)
        + b1_ref[0, 0], 0.0)
    h = jnp.maximum(
        jnp.dot(h, w2_ref[0], preferred_element_type=jnp.float32)
        + b2_ref[0, 0], 0.0)
    o = (jnp.dot(h, w3_ref[0], preferred_element_type=jnp.float32)
         + b3_ref[0, 0])
    o_ref[...] = jax.nn.sigmoid(o)


_mm_grid_spec = pltpu.PrefetchScalarGridSpec(
    num_scalar_prefetch=1,
    grid=(NB,),
    in_specs=[
        pl.BlockSpec((1, BLK, 1), lambda j, be: (j, 0, 0)),
        pl.BlockSpec((B, LATENT), lambda j, be: (0, 0)),
        pl.BlockSpec((1, LATENT, HIDDEN), lambda j, be: (be[j], 0, 0)),
        pl.BlockSpec((1, 1, HIDDEN), lambda j, be: (be[j], 0, 0)),
        pl.BlockSpec((1, HIDDEN, HIDDEN), lambda j, be: (be[j], 0, 0)),
        pl.BlockSpec((1, 1, HIDDEN), lambda j, be: (be[j], 0, 0)),
        pl.BlockSpec((1, HIDDEN, OUT), lambda j, be: (be[j], 0, 0)),
        pl.BlockSpec((1, 1, OUT), lambda j, be: (be[j], 0, 0)),
    ],
    out_specs=pl.BlockSpec((BLK, OUT), lambda j, be: (j, 0)),
)

_mm_call = pl.pallas_call(
    _mm_body,
    grid_spec=_mm_grid_spec,
    out_shape=jax.ShapeDtypeStruct((R, OUT), jnp.float32),
)


def _route(code):
    code = code.astype(jnp.int32)
    oh = (code[:, None] == jnp.arange(E, dtype=jnp.int32)[None, :]).astype(
        jnp.int32)
    counts = jnp.sum(oh, axis=0)
    rank = jnp.take_along_axis(jnp.cumsum(oh, axis=0) - oh, code[:, None],
                               axis=1)[:, 0]
    nblk = (counts + BLK - 1) // BLK
    bstart = jnp.cumsum(nblk) - nblk
    total = jnp.sum(nblk)
    j = jnp.arange(NB, dtype=jnp.int32)
    e_all = jnp.sum(
        (j[:, None] >= bstart[None, :]).astype(jnp.int32), axis=1) - 1
    last_e = e_all[jnp.clip(total - 1, 0, NB - 1)]
    be = jnp.where(j < total, e_all, last_e).astype(jnp.int32)
    pos = (bstart[code] * BLK + rank).astype(jnp.int32)
    gidx = jnp.zeros((R,), jnp.int32).at[pos].set(
        jnp.arange(B, dtype=jnp.int32))
    return be, gidx, pos


def kernel(input, code, W1, b1, W2, b2, W3, b3):
    sc_unsort = _sc_kernels()
    be, gidx, pos = _route(code)
    y = _mm_call(be, gidx.reshape(NB, BLK, 1), input,
                 W1, b1.reshape(E, 1, HIDDEN), W2,
                 b2.reshape(E, 1, HIDDEN), W3, b3.reshape(E, 1, OUT))
    out = sc_unsort(y, pos)
    return out.reshape((B,) + RES)


# speedup vs baseline: 1.1998x; 1.1998x over previous
import functools

import jax
import jax.numpy as jnp
from jax import lax
from jax.experimental import pallas as pl
from jax.experimental.pallas import tpu as pltpu
from jax.experimental.pallas import tpu_sc as plsc

E = 8
B = 2048
LATENT = 128
HIDDEN = 512
RES = (3, 32, 32)
OUT = RES[0] * RES[1] * RES[2]

BLK = 256
NB = B // BLK + E
R = NB * BLK
NW = 32
GROWS = R // NW
UCH = 16
UK = B // (NW * UCH)


def _wid():
    return lax.axis_index("s") * 2 + lax.axis_index("c")


@functools.lru_cache(maxsize=1)
def _sc_kernels():
    mesh = plsc.VectorSubcoreMesh(core_axis_name="c", subcore_axis_name="s")

    @functools.partial(
        pl.kernel,
        mesh=mesh,
        out_type=jax.ShapeDtypeStruct((B, OUT), jnp.float32),
        scratch_types=[
            pltpu.VMEM((UCH * UK,), jnp.int32),
            pltpu.VMEM((UCH, OUT), jnp.float32),
            pltpu.VMEM((UCH, OUT), jnp.float32),
            pltpu.SemaphoreType.DMA,
            pltpu.SemaphoreType.DMA,
        ],
    )
    def sc_unsort(y_hbm, pos_hbm, out_hbm, idx_v, buf0, buf1, sem0, sem1):
        base = _wid() * (UCH * UK)
        pltpu.sync_copy(pos_hbm.at[pl.ds(base, UCH * UK)], idx_v)
        bufs = (buf0, buf1)
        sems = (sem0, sem1)
        cps = [None, None]
        for k in range(UK):
            cps[k % 2] = pltpu.async_copy(
                y_hbm.at[idx_v.at[pl.ds(k * UCH, UCH)]], bufs[k % 2],
                sems[k % 2])
            if k > 0:
                cps[(k - 1) % 2].wait()
                pltpu.sync_copy(
                    bufs[(k - 1) % 2],
                    out_hbm.at[pl.ds(base + (k - 1) * UCH, UCH)])
        cps[(UK - 1) % 2].wait()
        pltpu.sync_copy(bufs[(UK - 1) % 2],
                        out_hbm.at[pl.ds(base + (UK - 1) * UCH, UCH)])

    return sc_unsort


def _mm_body(be_ref, gidx_ref, x_ref, w1_ref, b1_ref, w2_ref, b2_ref, w3_ref,
             b3_ref, o_ref):
    gi = gidx_ref[0]
    sel = (lax.broadcasted_iota(jnp.int32, (BLK, B), 1) == gi)
    x = jnp.dot(sel.astype(jnp.float32), x_ref[...],
                preferred_element_type=jnp.float32)
    h = jnp.maximum(
        jnp.dot(x, w1_ref[0], preferred_element_type=jnp.float32)
        + b1_ref[0, 0], 0.0)
    h = jnp.maximum(
        jnp.dot(h, w2_ref[0], preferred_element_type=jnp.float32)
        + b2_ref[0, 0], 0.0)
    o = (jnp.dot(h, w3_ref[0], preferred_element_type=jnp.float32)
         + b3_ref[0, 0])
    o_ref[...] = jax.nn.sigmoid(o)


_mm_grid_spec = pltpu.PrefetchScalarGridSpec(
    num_scalar_prefetch=1,
    grid=(NB,),
    in_specs=[
        pl.BlockSpec((1, BLK, 1), lambda j, be: (j, 0, 0)),
        pl.BlockSpec((B, LATENT), lambda j, be: (0, 0)),
        pl.BlockSpec((1, LATENT, HIDDEN), lambda j, be: (be[j], 0, 0)),
        pl.BlockSpec((1, 1, HIDDEN), lambda j, be: (be[j], 0, 0)),
        pl.BlockSpec((1, HIDDEN, HIDDEN), lambda j, be: (be[j], 0, 0)),
        pl.BlockSpec((1, 1, HIDDEN), lambda j, be: (be[j], 0, 0)),
        pl.BlockSpec((1, HIDDEN, OUT), lambda j, be: (be[j], 0, 0)),
        pl.BlockSpec((1, 1, OUT), lambda j, be: (be[j], 0, 0)),
    ],
    out_specs=pl.BlockSpec((BLK, OUT), lambda j, be: (j, 0)),
)

_mm_call = pl.pallas_call(
    _mm_body,
    grid_spec=_mm_grid_spec,
    out_shape=jax.ShapeDtypeStruct((R, OUT), jnp.float32),
)


def _route(code):
    code = code.astype(jnp.int32)
    ohT = (jnp.arange(E, dtype=jnp.int32)[:, None] == code[None, :]).astype(
        jnp.int32)
    counts = jnp.sum(ohT, axis=1)
    rank = jnp.sum(ohT * (jnp.cumsum(ohT, axis=1) - 1), axis=0)
    nblk = (counts + BLK - 1) // BLK
    bstart = jnp.cumsum(nblk) - nblk
    total = jnp.sum(nblk)
    j = jnp.arange(NB, dtype=jnp.int32)
    e_all = jnp.sum(
        (j[:, None] >= bstart[None, :]).astype(jnp.int32), axis=1) - 1
    last_e = e_all[jnp.clip(total - 1, 0, NB - 1)]
    be = jnp.where(j < total, e_all, last_e).astype(jnp.int32)
    pos = (jnp.sum(ohT * bstart[:, None], axis=0) * BLK + rank).astype(
        jnp.int32)
    gidx = jnp.zeros((R,), jnp.int32).at[pos].set(
        jnp.arange(B, dtype=jnp.int32))
    return be, gidx, pos


def kernel(input, code, W1, b1, W2, b2, W3, b3):
    sc_unsort = _sc_kernels()
    be, gidx, pos = _route(code)
    y = _mm_call(be, gidx.reshape(NB, BLK, 1), input,
                 W1, b1.reshape(E, 1, HIDDEN), W2,
                 b2.reshape(E, 1, HIDDEN), W3, b3.reshape(E, 1, OUT))
    out = sc_unsort(y, pos)
    return out.reshape((B,) + RES)


# speedup vs baseline: 1.3037x; 1.0866x over previous
import functools

import jax
import jax.numpy as jnp
from jax import lax
from jax.experimental import pallas as pl
from jax.experimental.pallas import tpu as pltpu
from jax.experimental.pallas import tpu_sc as plsc

E = 8
B = 2048
LATENT = 128
HIDDEN = 512
RES = (3, 32, 32)
OUT = RES[0] * RES[1] * RES[2]

BLK = 256
NB = B // BLK + E
R = NB * BLK
NW = 32
GROWS = R // NW
UCH = 16
UK = B // (NW * UCH)


def _wid():
    return lax.axis_index("s") * 2 + lax.axis_index("c")


@functools.lru_cache(maxsize=1)
def _sc_kernels():
    mesh = plsc.VectorSubcoreMesh(core_axis_name="c", subcore_axis_name="s")

    @functools.partial(
        pl.kernel,
        mesh=mesh,
        out_type=jax.ShapeDtypeStruct((B, OUT), jnp.float32),
        scratch_types=[
            pltpu.VMEM((UCH * UK,), jnp.int32),
            pltpu.VMEM((UCH, OUT), jnp.float32),
            pltpu.VMEM((UCH, OUT), jnp.float32),
            pltpu.SemaphoreType.DMA,
            pltpu.SemaphoreType.DMA,
        ],
    )
    def sc_unsort(y_hbm, pos_hbm, out_hbm, idx_v, buf0, buf1, sem0, sem1):
        base = _wid() * (UCH * UK)
        pltpu.sync_copy(pos_hbm.at[pl.ds(base, UCH * UK)], idx_v)
        bufs = (buf0, buf1)
        sems = (sem0, sem1)
        cps = [None, None]
        for k in range(UK):
            cps[k % 2] = pltpu.async_copy(
                y_hbm.at[idx_v.at[pl.ds(k * UCH, UCH)]], bufs[k % 2],
                sems[k % 2])
            if k > 0:
                cps[(k - 1) % 2].wait()
                pltpu.sync_copy(
                    bufs[(k - 1) % 2],
                    out_hbm.at[pl.ds(base + (k - 1) * UCH, UCH)])
        cps[(UK - 1) % 2].wait()
        pltpu.sync_copy(bufs[(UK - 1) % 2],
                        out_hbm.at[pl.ds(base + (UK - 1) * UCH, UCH)])

    return sc_unsort


def _mm_body(be_ref, pos_ref, x_ref, w1_ref, b1_ref, w2_ref, b2_ref, w3_ref,
             b3_ref, o_ref):
    j = pl.program_id(0)
    sel = (lax.broadcasted_iota(jnp.int32, (BLK, B), 0)
           == pos_ref[0] - j * BLK)
    x = jnp.dot(sel.astype(jnp.float32), x_ref[...],
                preferred_element_type=jnp.float32)
    h = jnp.maximum(
        jnp.dot(x, w1_ref[0], preferred_element_type=jnp.float32)
        + b1_ref[0, 0], 0.0)
    h = jnp.maximum(
        jnp.dot(h, w2_ref[0], preferred_element_type=jnp.float32)
        + b2_ref[0, 0], 0.0)
    o = (jnp.dot(h, w3_ref[0], preferred_element_type=jnp.float32)
         + b3_ref[0, 0])
    o_ref[...] = jax.nn.sigmoid(o)


_mm_grid_spec = pltpu.PrefetchScalarGridSpec(
    num_scalar_prefetch=1,
    grid=(NB,),
    in_specs=[
        pl.BlockSpec((1, 1, B), lambda j, be: (0, 0, 0)),
        pl.BlockSpec((B, LATENT), lambda j, be: (0, 0)),
        pl.BlockSpec((1, LATENT, HIDDEN), lambda j, be: (be[j], 0, 0)),
        pl.BlockSpec((1, 1, HIDDEN), lambda j, be: (be[j], 0, 0)),
        pl.BlockSpec((1, HIDDEN, HIDDEN), lambda j, be: (be[j], 0, 0)),
        pl.BlockSpec((1, 1, HIDDEN), lambda j, be: (be[j], 0, 0)),
        pl.BlockSpec((1, HIDDEN, OUT), lambda j, be: (be[j], 0, 0)),
        pl.BlockSpec((1, 1, OUT), lambda j, be: (be[j], 0, 0)),
    ],
    out_specs=pl.BlockSpec((BLK, OUT), lambda j, be: (j, 0)),
)

_mm_call = pl.pallas_call(
    _mm_body,
    grid_spec=_mm_grid_spec,
    out_shape=jax.ShapeDtypeStruct((R, OUT), jnp.float32),
)


def _route(code):
    code = code.astype(jnp.int32)
    ohT = (jnp.arange(E, dtype=jnp.int32)[:, None] == code[None, :]).astype(
        jnp.int32)
    counts = jnp.sum(ohT, axis=1)
    rank = jnp.sum(ohT * (jnp.cumsum(ohT, axis=1) - 1), axis=0)
    nblk = (counts + BLK - 1) // BLK
    bstart = jnp.cumsum(nblk) - nblk
    total = jnp.sum(nblk)
    j = jnp.arange(NB, dtype=jnp.int32)
    e_all = jnp.sum(
        (j[:, None] >= bstart[None, :]).astype(jnp.int32), axis=1) - 1
    last_e = e_all[jnp.clip(total - 1, 0, NB - 1)]
    be = jnp.where(j < total, e_all, last_e).astype(jnp.int32)
    pos = (jnp.sum(ohT * bstart[:, None], axis=0) * BLK + rank).astype(
        jnp.int32)
    return be, pos


def kernel(input, code, W1, b1, W2, b2, W3, b3):
    sc_unsort = _sc_kernels()
    be, pos = _route(code)
    y = _mm_call(be, pos.reshape(1, 1, B), input,
                 W1, b1.reshape(E, 1, HIDDEN), W2,
                 b2.reshape(E, 1, HIDDEN), W3, b3.reshape(E, 1, OUT))
    out = sc_unsort(y, pos)
    return out.reshape((B,) + RES)


# speedup vs baseline: 1.3295x; 1.0198x over previous
import functools

import jax
import jax.numpy as jnp
from jax import lax
from jax.experimental import pallas as pl
from jax.experimental.pallas import tpu as pltpu
from jax.experimental.pallas import tpu_sc as plsc

E = 8
B = 2048
LATENT = 128
HIDDEN = 512
RES = (3, 32, 32)
OUT = RES[0] * RES[1] * RES[2]

BLK = 256
NB = B // BLK + E
R = NB * BLK
NW = 32
GROWS = R // NW
UCH = 16
UK = B // (NW * UCH)


def _wid():
    return lax.axis_index("s") * 2 + lax.axis_index("c")


@functools.lru_cache(maxsize=1)
def _sc_kernels():
    mesh = plsc.VectorSubcoreMesh(core_axis_name="c", subcore_axis_name="s")

    @functools.partial(
        pl.kernel,
        mesh=mesh,
        out_type=jax.ShapeDtypeStruct((B, OUT), jnp.float32),
        scratch_types=[
            pltpu.VMEM((UCH * UK,), jnp.int32),
            pltpu.VMEM((UCH, OUT), jnp.float32),
            pltpu.VMEM((UCH, OUT), jnp.float32),
            pltpu.SemaphoreType.DMA,
            pltpu.SemaphoreType.DMA,
        ],
    )
    def sc_unsort(y_hbm, pos_hbm, out_hbm, idx_v, buf0, buf1, sem0, sem1):
        base = _wid() * (UCH * UK)
        pltpu.sync_copy(pos_hbm.at[pl.ds(base, UCH * UK)], idx_v)
        bufs = (buf0, buf1)
        sems = (sem0, sem1)
        cps = [None, None]
        for k in range(UK):
            cps[k % 2] = pltpu.async_copy(
                y_hbm.at[idx_v.at[pl.ds(k * UCH, UCH)]], bufs[k % 2],
                sems[k % 2])
            if k > 0:
                cps[(k - 1) % 2].wait()
                pltpu.sync_copy(
                    bufs[(k - 1) % 2],
                    out_hbm.at[pl.ds(base + (k - 1) * UCH, UCH)])
        cps[(UK - 1) % 2].wait()
        pltpu.sync_copy(bufs[(UK - 1) % 2],
                        out_hbm.at[pl.ds(base + (UK - 1) * UCH, UCH)])

    return sc_unsort


def _mm_body(be_ref, pos_ref, x_ref, w1_ref, b1_ref, w2_ref, b2_ref, w3_ref,
             b3_ref, o_ref):
    j = pl.program_id(0)

    @pl.when(j < be_ref[NB])
    def _():
        sel = (lax.broadcasted_iota(jnp.int32, (BLK, B), 0)
               == pos_ref[0] - j * BLK)
        x = jnp.dot(sel.astype(jnp.float32), x_ref[...],
                    preferred_element_type=jnp.float32)
        h = jnp.maximum(
            jnp.dot(x, w1_ref[0], preferred_element_type=jnp.float32)
            + b1_ref[0, 0], 0.0)
        h = jnp.maximum(
            jnp.dot(h, w2_ref[0], preferred_element_type=jnp.float32)
            + b2_ref[0, 0], 0.0)
        o = (jnp.dot(h, w3_ref[0], preferred_element_type=jnp.float32)
             + b3_ref[0, 0])
        o_ref[...] = jax.nn.sigmoid(o)


_mm_grid_spec = pltpu.PrefetchScalarGridSpec(
    num_scalar_prefetch=1,
    grid=(NB,),
    in_specs=[
        pl.BlockSpec((1, 1, B), lambda j, be: (0, 0, 0)),
        pl.BlockSpec((B, LATENT), lambda j, be: (0, 0)),
        pl.BlockSpec((1, LATENT, HIDDEN), lambda j, be: (be[j], 0, 0)),
        pl.BlockSpec((1, 1, HIDDEN), lambda j, be: (be[j], 0, 0)),
        pl.BlockSpec((1, HIDDEN, HIDDEN), lambda j, be: (be[j], 0, 0)),
        pl.BlockSpec((1, 1, HIDDEN), lambda j, be: (be[j], 0, 0)),
        pl.BlockSpec((1, HIDDEN, OUT), lambda j, be: (be[j], 0, 0)),
        pl.BlockSpec((1, 1, OUT), lambda j, be: (be[j], 0, 0)),
    ],
    out_specs=pl.BlockSpec((BLK, OUT), lambda j, be: (j, 0)),
)

_mm_call = pl.pallas_call(
    _mm_body,
    grid_spec=_mm_grid_spec,
    out_shape=jax.ShapeDtypeStruct((R, OUT), jnp.float32),
)


def _route(code):
    code = code.astype(jnp.int32)
    ohT = (jnp.arange(E, dtype=jnp.int32)[:, None] == code[None, :]).astype(
        jnp.int32)
    counts = jnp.sum(ohT, axis=1)
    rank = jnp.sum(ohT * (jnp.cumsum(ohT, axis=1) - 1), axis=0)
    nblk = (counts + BLK - 1) // BLK
    bstart = jnp.cumsum(nblk) - nblk
    total = jnp.sum(nblk)
    j = jnp.arange(NB, dtype=jnp.int32)
    e_all = jnp.sum(
        (j[:, None] >= bstart[None, :]).astype(jnp.int32), axis=1) - 1
    last_e = e_all[jnp.clip(total - 1, 0, NB - 1)]
    be = jnp.where(j < total, e_all, last_e).astype(jnp.int32)
    be = jnp.concatenate([be, total[None].astype(jnp.int32)])
    pos = (jnp.sum(ohT * bstart[:, None], axis=0) * BLK + rank).astype(
        jnp.int32)
    return be, pos


def kernel(input, code, W1, b1, W2, b2, W3, b3):
    sc_unsort = _sc_kernels()
    be, pos = _route(code)
    y = _mm_call(be, pos.reshape(1, 1, B), input,
                 W1, b1.reshape(E, 1, HIDDEN), W2,
                 b2.reshape(E, 1, HIDDEN), W3, b3.reshape(E, 1, OUT))
    out = sc_unsort(y, pos)
    return out.reshape((B,) + RES)
